# Initial kernel scaffold; baseline (speedup 1.0000x reference)
import jax
import jax.numpy as jnp
from jax.experimental import pallas as pl


def kernel(feats, edge_index, W0, W1, W2):
    raise NotImplementedError("write your pallas kernel here")


# speedup vs baseline: 3.3726x; 3.3726x over previous
import functools

import jax
import jax.numpy as jnp
from jax import lax
from jax.experimental import pallas as pl
from jax.experimental.pallas import tpu as pltpu
from jax.experimental.pallas import tpu_sc as plsc

N = 10000
D = 128
OUT = 128

NC = 2
NS = 16
NW = NC * NS
CH = 128

N_PAD = 10240
ZCH = N_PAD // (NS * CH)
ROWS_PT = N_PAD // NS

BLK = 1024


def _sc_mesh():
    return plsc.VectorSubcoreMesh(core_axis_name="c", subcore_axis_name="s",
                                  num_cores=NC, num_subcores=NS)


def _sc_bincount_body(dst_hbm, ones_hbm, zeros_hbm, out_hbm, dst_v, ones_v, cnt_sp):
    c = lax.axis_index("c")
    s = lax.axis_index("s")
    w = s * NC + c
    E_pad = dst_hbm.shape[0]
    k_per_w = E_pad // (NW * CH)

    pltpu.sync_copy(zeros_hbm, ones_v)
    for z in range(ZCH):
        pltpu.sync_copy(ones_v, cnt_sp.at[pl.ds((s * ZCH + z) * CH, CH)])
    pltpu.sync_copy(ones_hbm, ones_v)
    plsc.subcore_barrier()

    ebase = w * k_per_w * CH

    def body(k, carry):
        pltpu.sync_copy(dst_hbm.at[pl.ds(ebase + k * CH, CH)], dst_v.at[0])
        pltpu.sync_copy(ones_v, cnt_sp.at[dst_v.at[0]], add=True)
        return carry

    lax.fori_loop(0, k_per_w, body, 0)
    plsc.subcore_barrier()
    pltpu.sync_copy(cnt_sp.at[pl.ds(s * ROWS_PT, ROWS_PT)],
                    out_hbm.at[c, pl.ds(s * ROWS_PT, ROWS_PT)])


def _sc_hop_body(g_hbm, src_hbm, dst_hbm, zeros_hbm, out_hbm,
                 src_v, dst_v, rows_v, acc_sp, sem):
    c = lax.axis_index("c")
    s = lax.axis_index("s")
    w = s * NC + c
    E_pad = src_hbm.shape[0]
    k_per_w = E_pad // (NW * CH)

    pltpu.sync_copy(zeros_hbm, rows_v)
    for z in range(ZCH):
        pltpu.sync_copy(rows_v, acc_sp.at[pl.ds((s * ZCH + z) * CH, CH)])
    plsc.subcore_barrier()

    ebase = w * k_per_w * CH

    def body(k, carry):
        e0 = ebase + k * CH
        pltpu.sync_copy(src_hbm.at[pl.ds(e0, CH)], src_v)
        pltpu.async_copy(g_hbm.at[src_v], rows_v, sem).wait()
        pltpu.sync_copy(dst_hbm.at[pl.ds(e0, CH)], dst_v.at[0])
        pltpu.sync_copy(rows_v, acc_sp.at[dst_v.at[0]], add=True)
        return carry

    lax.fori_loop(0, k_per_w, body, 0)
    plsc.subcore_barrier()
    pltpu.sync_copy(acc_sp.at[pl.ds(s * ROWS_PT, ROWS_PT)],
                    out_hbm.at[c, pl.ds(s * ROWS_PT, ROWS_PT)])


def _make_sc_bincount(interpret=False):
    return pl.kernel(
        _sc_bincount_body,
        mesh=_sc_mesh(),
        interpret=interpret,
        out_type=jax.ShapeDtypeStruct((NC, N_PAD, D), jnp.float32),
        scratch_types=[
            pltpu.VMEM((1, CH), jnp.int32),
            pltpu.VMEM((CH, D), jnp.float32),
            pltpu.VMEM_SHARED((N_PAD, D), jnp.float32),
        ],
    )


def _make_sc_hop(interpret=False):
    return pl.kernel(
        _sc_hop_body,
        mesh=_sc_mesh(),
        interpret=interpret,
        out_type=jax.ShapeDtypeStruct((NC, N_PAD, D), jnp.float32),
        scratch_types=[
            pltpu.VMEM((CH,), jnp.int32),
            pltpu.VMEM((1, CH), jnp.int32),
            pltpu.VMEM((CH, D), jnp.float32),
            pltpu.VMEM_SHARED((N_PAD, D), jnp.float32),
            pltpu.SemaphoreType.DMA,
        ],
    )


_sc_bincount = _make_sc_bincount()
_sc_hop = _make_sc_hop()


def _tc0_body(feats_ref, c0_ref, c1_ref, w_ref, out_ref, g_ref, nb_ref):
    pid = pl.program_id(0)
    deg = c0_ref[:, 0:1] + c1_ref[:, 0:1]
    nrm = lax.rsqrt(jnp.maximum(deg, 1.0))
    row = pid * BLK + lax.broadcasted_iota(jnp.int32, (BLK, 1), 0)
    nrm = jnp.where(row < N, nrm, 0.0)
    nb = jnp.broadcast_to(nrm, (BLK, D))
    h = feats_ref[...]
    out_ref[...] = jnp.dot(h, w_ref[...], preferred_element_type=jnp.float32)
    g_ref[...] = h * nb
    nb_ref[...] = nb


def _tc_hop_body(a0_ref, a1_ref, nb_ref, w_ref, out_ref, g_ref):
    nb = nb_ref[...]
    h = (a0_ref[...] + a1_ref[...]) * nb
    out_ref[...] = jnp.dot(h, w_ref[...], preferred_element_type=jnp.float32)
    g_ref[...] = h * nb


def _tc_fin_body(a0_ref, a1_ref, nb_ref, w_ref, out_ref):
    h = (a0_ref[...] + a1_ref[...]) * nb_ref[...]
    out_ref[...] = jnp.dot(h, w_ref[...], preferred_element_type=jnp.float32)


_GRID = (N_PAD // BLK,)
_ROWS = pl.BlockSpec((BLK, D), lambda i: (i, 0))
_CNTS = pl.BlockSpec((BLK, D), lambda i: (i, 0))
_WSPEC = pl.BlockSpec((D, OUT), lambda i: (0, 0))
_OUTS = pl.BlockSpec((BLK, OUT), lambda i: (i, 0))

_tc0 = pl.pallas_call(
    _tc0_body,
    grid=_GRID,
    in_specs=[_ROWS, _CNTS, _CNTS, _WSPEC],
    out_specs=[_OUTS, _ROWS, _ROWS],
    out_shape=[jax.ShapeDtypeStruct((N_PAD, OUT), jnp.float32),
               jax.ShapeDtypeStruct((N_PAD, D), jnp.float32),
               jax.ShapeDtypeStruct((N_PAD, D), jnp.float32)],
)

_tc_hop = pl.pallas_call(
    _tc_hop_body,
    grid=_GRID,
    in_specs=[_ROWS, _ROWS, _ROWS, _WSPEC],
    out_specs=[_OUTS, _ROWS],
    out_shape=[jax.ShapeDtypeStruct((N_PAD, OUT), jnp.float32),
               jax.ShapeDtypeStruct((N_PAD, D), jnp.float32)],
)

_tc_fin = pl.pallas_call(
    _tc_fin_body,
    grid=_GRID,
    in_specs=[_ROWS, _ROWS, _ROWS, _WSPEC],
    out_specs=_OUTS,
    out_shape=jax.ShapeDtypeStruct((N_PAD, OUT), jnp.float32),
)


def kernel(feats, edge_index, W0, W1, W2):
    E = edge_index.shape[1]
    ew = NW * CH
    e_pad = ((E + ew - 1) // ew) * ew
    pad = e_pad - E

    src = jnp.concatenate([edge_index[0], jnp.full((pad,), N, jnp.int32)])
    dst = jnp.concatenate([edge_index[1], jnp.full((pad,), N, jnp.int32)])
    feats_p = jnp.zeros((N_PAD, D), jnp.float32).at[:N].set(feats)

    ones_rows = jnp.ones((CH, D), jnp.float32)
    zrows = jnp.zeros((CH, D), jnp.float32)

    counts = _sc_bincount(dst, ones_rows, zrows)
    out0, g0, nb = _tc0(feats_p, counts[0], counts[1], W0)
    acc1 = _sc_hop(g0, src, dst, zrows)
    out1, g1 = _tc_hop(acc1[0], acc1[1], nb, W1)
    acc2 = _sc_hop(g1, src, dst, zrows)
    out2 = _tc_fin(acc2[0], acc2[1], nb, W2)

    return jnp.concatenate([out0[:N], out1[:N], out2[:N]], axis=1)
